# baseline (speedup 1.0000x reference)
import jax
import jax.numpy as jnp
from jax import lax
from jax.experimental import pallas as pl
from jax.experimental.pallas import tpu as pltpu
from jax.experimental.pallas import tpu_sc as plsc

_A = 8
_G = 256
_D = 16
_B = 1024
_NW = 32
_GPW = _G // _NW
_NP = 8
_QB = _B // _NP
_NCH = _QB // 16
_TSZ = 1 << _A
_GSZ = _TSZ * _D

_NET = [(0, 1), (2, 3), (4, 5), (6, 7),
        (0, 2), (1, 3), (4, 6), (5, 7),
        (1, 2), (5, 6), (0, 4), (3, 7),
        (1, 5), (2, 6),
        (1, 4), (3, 6),
        (2, 4), (3, 5),
        (3, 4)]


def _sc_body(xt_hbm, p_hbm, out_hbm, xn_v, x_t, t_v, o_v, t_sem, o_sem):
    wid = lax.axis_index("c") * 16 + lax.axis_index("s")

    pconst = [jnp.full((16,), (1 << k) * _D, jnp.int32) for k in range(_A)]
    iota = lax.iota(jnp.int32, 16)

    tcps = [pltpu.async_copy(p_hbm.at[wid * _GPW + gi2],
                             t_v.at[pl.ds(gi2 * _GSZ, _GSZ)], t_sem)
            for gi2 in range(_GPW)]
    for cp in tcps:
        cp.wait()

    def out_win(q):
        return out_hbm.at[pl.ds(q * _QB, _QB),
                          pl.ds(wid * _GPW * _D, _GPW * _D)]

    xblk = (wid // 2) * 128
    half = (wid % 2) * 64
    iota129 = iota * 129

    def quarter_body(q, carry):
        par = q % 2
        pltpu.sync_copy(
            xt_hbm.at[pl.ds(q * _QB, _QB), pl.ds(xblk, 128)], xn_v)

        @pl.when(q >= 2)
        def _():
            pltpu.make_async_copy(o_v.at[par], out_win(q - 2), o_sem).wait()

        def trans_body(bb, c2):
            bvec = iota129 + jnp.full((16,), bb * 16, jnp.int32)
            for blk in range(4):
                for j in range(16):
                    b = bb * 16 + j
                    vrow = xn_v[b, pl.ds(half + blk * 16, 16)]
                    idx = bvec + jnp.full((16,), blk * 16 * 129 + j,
                                          jnp.int32)
                    plsc.store_scatter(x_t, [idx], vrow)
            return c2

        lax.fori_loop(0, _QB // 16, trans_body, 0)

        def group_body(gi, c2):
            mask_init = jnp.full((16,), 255 * _D, jnp.int32) + jnp.full(
                (16,), gi * _GSZ, jnp.int32)
            row0 = plsc.load_gather(t_v, [mask_init + iota])

            def sort_phase(c):
                v = [x_t[pl.ds((gi * _A + k) * 129 + c * 16, 16)]
                     for k in range(_A)]
                p = list(pconst)

                for a, b in _NET:
                    le = v[a] <= v[b]
                    va = jnp.minimum(v[a], v[b])
                    vb = jnp.maximum(v[a], v[b])
                    pa = jnp.where(le, p[a], p[b])
                    pb = jnp.where(le, p[b], p[a])
                    v[a], v[b], p[a], p[b] = va, vb, pa, pb

                mask = mask_init
                masks = [None] * _A
                coefs = [None] * _A
                for r in range(_A):
                    masks[r] = mask
                    coefs[r] = v[r] if r == 0 else v[r] - v[r - 1]
                    if r < _A - 1:
                        mask = mask - p[r]
                return tuple(masks), tuple(coefs)

            def gather_phase(c, masks, coefs):
                base = c * 16
                for p0 in range(0, 16, 4):
                    rows = {}
                    for pp in range(p0, p0 + 4):
                        for r in range(1, _A):
                            idx = jnp.full((16,), masks[r][pp],
                                           jnp.int32) + iota
                            rows[pp, r] = plsc.load_gather(t_v, [idx])
                    for pp in range(p0, p0 + 4):
                        terms = [row0 * coefs[0][pp]]
                        terms += [rows[pp, r] * coefs[r][pp]
                                  for r in range(1, _A)]
                        while len(terms) > 1:
                            terms = [terms[i] + terms[i + 1]
                                     for i in range(0, len(terms), 2)]
                        o_v[par, base + pp, pl.ds(gi * _D, _D)] = terms[0]

            def chunk_body(c, carry):
                masks, coefs = sort_phase(c)
                gather_phase(c, masks, coefs)
                return carry

            lax.fori_loop(0, _NCH, chunk_body, 0)
            return c2

        lax.fori_loop(0, _GPW, group_body, 0)
        pltpu.async_copy(o_v.at[par], out_win(q), o_sem)
        return carry

    lax.fori_loop(0, _NP, quarter_body, 0)
    for j in (_NP - 2, _NP - 1):
        pltpu.make_async_copy(o_v.at[j % 2], out_win(j), o_sem).wait()


_sc_call = pl.kernel(
    _sc_body,
    out_type=jax.ShapeDtypeStruct((_B, _G * _D), jnp.float32),
    mesh=plsc.VectorSubcoreMesh(core_axis_name="c", subcore_axis_name="s"),
    compiler_params=pltpu.CompilerParams(needs_layout_passes=False),
    scratch_types=[
        pltpu.VMEM((_QB, 128), jnp.float32),
        pltpu.VMEM((_GPW * _A * 129,), jnp.float32),
        pltpu.VMEM((_GPW * _GSZ,), jnp.float32),
        pltpu.VMEM((2, _QB, _GPW * _D), jnp.float32),
        pltpu.SemaphoreType.DMA,
        pltpu.SemaphoreType.DMA,
    ],
)


def kernel(X, params):
    return _sc_call(X, params.reshape(_G, _GSZ))


# speedup vs baseline: 1.1030x; 1.1030x over previous
import jax
import jax.numpy as jnp
from jax import lax
from jax.experimental import pallas as pl
from jax.experimental.pallas import tpu as pltpu
from jax.experimental.pallas import tpu_sc as plsc

_A = 8
_G = 256
_D = 16
_B = 1024
_NW = 32
_GPW = _G // _NW
_NP = 8
_QB = _B // _NP
_NCH = _QB // 16
_TSZ = 1 << _A
_GSZ = _TSZ * _D

_NET = [(0, 1), (2, 3), (4, 5), (6, 7),
        (0, 2), (1, 3), (4, 6), (5, 7),
        (1, 2), (5, 6), (0, 4), (3, 7),
        (1, 5), (2, 6),
        (1, 4), (3, 6),
        (2, 4), (3, 5),
        (3, 4)]


def _sc_body(xt_hbm, p_hbm, out_hbm, x_v, t_v, o_v, t_sem, o_sem):
    wid = lax.axis_index("c") * 16 + lax.axis_index("s")

    pconst = [jnp.full((16,), (1 << k) * _D, jnp.int32) for k in range(_A)]
    iota = lax.iota(jnp.int32, 16)

    tcps = [pltpu.async_copy(p_hbm.at[wid * _GPW + gi2],
                             t_v.at[pl.ds(gi2 * _GSZ, _GSZ)], t_sem)
            for gi2 in range(_GPW)]
    for cp in tcps:
        cp.wait()

    def out_win(q):
        return out_hbm.at[pl.ds(q * _QB, _QB),
                          pl.ds(wid * _GPW * _D, _GPW * _D)]

    def quarter_body(q, carry):
        par = q % 2
        pltpu.sync_copy(
            xt_hbm.at[pl.ds(wid * _GPW, _GPW), :, pl.ds(q * _QB, _QB)],
            x_v)

        @pl.when(q >= 2)
        def _():
            pltpu.make_async_copy(o_v.at[par], out_win(q - 2), o_sem).wait()

        def group_body(gi, c2):
            mask_init = jnp.full((16,), 255 * _D, jnp.int32) + jnp.full(
                (16,), gi * _GSZ, jnp.int32)
            row0 = plsc.load_gather(t_v, [mask_init + iota])

            def sort_phase(c):
                v = [x_v[gi, k, pl.ds(c * 16, 16)] for k in range(_A)]
                p = list(pconst)

                for a, b in _NET:
                    le = v[a] <= v[b]
                    va = jnp.minimum(v[a], v[b])
                    vb = jnp.maximum(v[a], v[b])
                    pa = jnp.where(le, p[a], p[b])
                    pb = jnp.where(le, p[b], p[a])
                    v[a], v[b], p[a], p[b] = va, vb, pa, pb

                mask = mask_init
                masks = [None] * _A
                coefs = [None] * _A
                for r in range(_A):
                    masks[r] = mask
                    coefs[r] = v[r] if r == 0 else v[r] - v[r - 1]
                    if r < _A - 1:
                        mask = mask - p[r]
                return tuple(masks), tuple(coefs)

            def gather_phase(c, masks, coefs):
                base = c * 16
                for p0 in range(0, 16, 4):
                    rows = {}
                    for pp in range(p0, p0 + 4):
                        for r in range(1, _A):
                            idx = jnp.full((16,), masks[r][pp],
                                           jnp.int32) + iota
                            rows[pp, r] = plsc.load_gather(t_v, [idx])
                    for pp in range(p0, p0 + 4):
                        terms = [row0 * coefs[0][pp]]
                        terms += [rows[pp, r] * coefs[r][pp]
                                  for r in range(1, _A)]
                        while len(terms) > 1:
                            terms = [terms[i] + terms[i + 1]
                                     for i in range(0, len(terms), 2)]
                        o_v[par, base + pp, pl.ds(gi * _D, _D)] = terms[0]

            def chunk_body(c, carry):
                masks, coefs = sort_phase(c)
                gather_phase(c, masks, coefs)
                return carry

            lax.fori_loop(0, _NCH, chunk_body, 0)
            return c2

        lax.fori_loop(0, _GPW, group_body, 0)
        pltpu.async_copy(o_v.at[par], out_win(q), o_sem)
        return carry

    lax.fori_loop(0, _NP, quarter_body, 0)
    for j in (_NP - 2, _NP - 1):
        pltpu.make_async_copy(o_v.at[j % 2], out_win(j), o_sem).wait()


_sc_call = pl.kernel(
    _sc_body,
    out_type=jax.ShapeDtypeStruct((_B, _G * _D), jnp.float32),
    mesh=plsc.VectorSubcoreMesh(core_axis_name="c", subcore_axis_name="s"),
    compiler_params=pltpu.CompilerParams(needs_layout_passes=False),
    scratch_types=[
        pltpu.VMEM((_GPW, _A, _QB), jnp.float32),
        pltpu.VMEM((_GPW * _GSZ,), jnp.float32),
        pltpu.VMEM((2, _QB, _GPW * _D), jnp.float32),
        pltpu.SemaphoreType.DMA,
        pltpu.SemaphoreType.DMA,
    ],
)


def kernel(X, params):
    xt = X.reshape(_B, _G, _A).transpose(1, 2, 0)
    return _sc_call(xt, params.reshape(_G, _GSZ))
